# baseline (speedup 1.0000x reference)
import functools

import jax
import jax.numpy as jnp
from jax import lax
from jax.experimental import pallas as pl
from jax.experimental.pallas import tpu as pltpu
from jax.experimental.pallas import tpu_sc as plsc

VOCAB = 457
EMB = 64
BATCH = 16384
SEQ_LEN = 200
N = BATCH * SEQ_LEN
LINES = N // 2
PIDX_ROWS = LINES // 128

_info = plsc.get_sparse_core_info()
NC = _info.num_cores
NS = _info.num_subcores
NW = NC * NS
SUB = 128
NSUB = 2
CHUNK_LINES = SUB * NSUB
PER_W_LINES = LINES // NW
N_ITERS = PER_W_LINES // CHUNK_LINES

assert LINES % (NW * CHUNK_LINES) == 0 and N_ITERS % 2 == 0


def _sc_lookup_one(pidx, pair_table):
    mesh = plsc.VectorSubcoreMesh(core_axis_name="c", subcore_axis_name="s")

    @functools.partial(
        pl.kernel,
        mesh=mesh,
        out_type=jax.ShapeDtypeStruct((LINES, 128), jnp.float32),
        scratch_types=[
            pltpu.VMEM((2, 8, SUB), jnp.int32),
            pltpu.VMEM((2, CHUNK_LINES, 128), jnp.float32),
            [pltpu.SemaphoreType.DMA, pltpu.SemaphoreType.DMA],
            [pltpu.SemaphoreType.DMA, pltpu.SemaphoreType.DMA],
        ],
        compiler_params=pltpu.CompilerParams(use_tc_tiling_on_sc=True),
    )
    def k(idx_hbm, tab_hbm, out_hbm, idx_v, rows_v, gsem, osem):
        wid = lax.axis_index("s") * NC + lax.axis_index("c")
        base_irow = wid * (PER_W_LINES // SUB)
        base_line = wid * PER_W_LINES

        def fire_gathers(g, b):
            irow = base_irow + g * NSUB
            pltpu.sync_copy(idx_hbm.at[pl.ds(irow, NSUB)],
                            idx_v.at[b, pl.ds(0, NSUB)])
            for j in range(NSUB):
                pltpu.async_copy(
                    tab_hbm.at[idx_v.at[b, j]],
                    rows_v.at[b, pl.ds(j * SUB, SUB)],
                    gsem[b],
                )

        def wait_gathers(b):
            for j in range(NSUB):
                pltpu.make_async_copy(
                    tab_hbm.at[idx_v.at[b, j]],
                    rows_v.at[b, pl.ds(j * SUB, SUB)],
                    gsem[b],
                ).wait()

        def fire_out(g, b):
            line = base_line + g * CHUNK_LINES
            pltpu.async_copy(rows_v.at[b],
                             out_hbm.at[pl.ds(line, CHUNK_LINES)], osem[b])

        def wait_out(g, b):
            line = base_line + g * CHUNK_LINES
            pltpu.make_async_copy(rows_v.at[b],
                                  out_hbm.at[pl.ds(line, CHUNK_LINES)],
                                  osem[b]).wait()

        fire_gathers(0, 0)

        def step(g2, carry):
            for b in range(2):
                g = 2 * g2 + b
                nb2 = 1 - b

                @pl.when(g + 1 < N_ITERS)
                def _():
                    @pl.when(g >= 1)
                    def _():
                        wait_out(g - 1, nb2)
                    fire_gathers(g + 1, nb2)

                wait_gathers(b)
                fire_out(g, b)
            return carry

        lax.fori_loop(0, N_ITERS // 2, step, 0)
        wait_out(N_ITERS - 1, (N_ITERS - 1) % 2)
        wait_out(N_ITERS - 2, (N_ITERS - 2) % 2)

    return k(pidx, pair_table)


def _pair_idx(a):
    p = a.astype(jnp.int32).reshape(LINES, 2)
    return (p[:, 0] * VOCAB + p[:, 1]).reshape(PIDX_ROWS, 128)


def _pair_table(tab):
    left = jnp.broadcast_to(tab[:, None, :], (VOCAB, VOCAB, EMB))
    right = jnp.broadcast_to(tab[None, :, :], (VOCAB, VOCAB, EMB))
    return jnp.concatenate([left, right], axis=-1).reshape(VOCAB * VOCAB, 128)


def _unlines(lines):
    return jnp.minimum(lines, jnp.float32(3.0e38)).reshape(BATCH, SEQ_LEN, EMB)


def kernel(seqs, exps, table_seq, table_exp):
    lines1 = _sc_lookup_one(_pair_idx(seqs), _pair_table(table_seq))
    lines2 = _sc_lookup_one(_pair_idx(exps), _pair_table(table_exp))
    return (_unlines(lines1), _unlines(lines2))


# speedup vs baseline: 1.3621x; 1.3621x over previous
import functools

import jax
import jax.numpy as jnp
from jax import lax
from jax.experimental import pallas as pl
from jax.experimental.pallas import tpu as pltpu
from jax.experimental.pallas import tpu_sc as plsc

VOCAB = 457
EMB = 64
BATCH = 16384
SEQ_LEN = 200
N = BATCH * SEQ_LEN
LINES = N // 2
PIDX_ROWS = LINES // 128

_info = plsc.get_sparse_core_info()
NC = _info.num_cores
NS = _info.num_subcores
NW = NC * NS
SUB = 128
NSUB = 2
CHUNK_LINES = SUB * NSUB
PER_W_LINES = LINES // NW
N_ITERS = PER_W_LINES // CHUNK_LINES

assert LINES % (NW * CHUNK_LINES) == 0 and N_ITERS % 2 == 0


def _sc_lookup_one(pidx, pair_table):
    mesh = plsc.VectorSubcoreMesh(core_axis_name="c", subcore_axis_name="s")

    @functools.partial(
        pl.kernel,
        mesh=mesh,
        out_type=jax.ShapeDtypeStruct((LINES, 128), jnp.float32),
        scratch_types=[
            pltpu.VMEM((2, 8, SUB), jnp.int32),
            pltpu.VMEM((2, CHUNK_LINES, 128), jnp.float32),
            [pltpu.SemaphoreType.DMA, pltpu.SemaphoreType.DMA],
            [pltpu.SemaphoreType.DMA, pltpu.SemaphoreType.DMA],
        ],
        compiler_params=pltpu.CompilerParams(use_tc_tiling_on_sc=True),
    )
    def k(idx_hbm, tab_hbm, out_hbm, idx_v, rows_v, gsem, osem):
        wid = lax.axis_index("s") * NC + lax.axis_index("c")
        base_irow = wid * (PER_W_LINES // SUB)
        base_line = wid * PER_W_LINES

        def fire_gathers(g, b):
            irow = base_irow + g * NSUB
            pltpu.sync_copy(idx_hbm.at[pl.ds(irow, NSUB)],
                            idx_v.at[b, pl.ds(0, NSUB)])
            for j in range(NSUB):
                pltpu.async_copy(
                    tab_hbm.at[idx_v.at[b, j]],
                    rows_v.at[b, pl.ds(j * SUB, SUB)],
                    gsem[b],
                )

        def wait_gathers(b):
            for j in range(NSUB):
                pltpu.make_async_copy(
                    tab_hbm.at[idx_v.at[b, j]],
                    rows_v.at[b, pl.ds(j * SUB, SUB)],
                    gsem[b],
                ).wait()

        def fire_out(g, b):
            line = base_line + g * CHUNK_LINES
            pltpu.async_copy(rows_v.at[b],
                             out_hbm.at[pl.ds(line, CHUNK_LINES)], osem[b])

        def wait_out(g, b):
            line = base_line + g * CHUNK_LINES
            pltpu.make_async_copy(rows_v.at[b],
                                  out_hbm.at[pl.ds(line, CHUNK_LINES)],
                                  osem[b]).wait()

        fire_gathers(0, 0)

        def step(g2, carry):
            for b in range(2):
                g = 2 * g2 + b
                nb2 = 1 - b

                @pl.when(g + 1 < N_ITERS)
                def _():
                    @pl.when(g >= 1)
                    def _():
                        wait_out(g - 1, nb2)
                    fire_gathers(g + 1, nb2)

                wait_gathers(b)
                fire_out(g, b)
            return carry

        lax.fori_loop(0, N_ITERS // 2, step, 0)
        wait_out(N_ITERS - 1, (N_ITERS - 1) % 2)
        wait_out(N_ITERS - 2, (N_ITERS - 2) % 2)

    return k(pidx, pair_table)


def _pair_idx(a):
    p = a.astype(jnp.int32).reshape(LINES, 2)
    return (p[:, 0] * VOCAB + p[:, 1]).reshape(PIDX_ROWS, 128)


def _pair_table(tab):
    left = jnp.broadcast_to(tab[:, None, :], (VOCAB, VOCAB, EMB))
    right = jnp.broadcast_to(tab[None, :, :], (VOCAB, VOCAB, EMB))
    return jnp.concatenate([left, right], axis=-1).reshape(VOCAB * VOCAB, 128)


_TR_BR = 1024
_TR_BC = 1280
_KD = SEQ_LEN * EMB


def _tc_transpose(x):
    def body(x_ref, y_ref):
        y_ref[...] = x_ref[...].T

    return pl.pallas_call(
        body,
        grid=(BATCH // _TR_BR, _KD // _TR_BC),
        in_specs=[pl.BlockSpec((_TR_BR, _TR_BC), lambda i, j: (i, j))],
        out_specs=pl.BlockSpec((_TR_BC, _TR_BR), lambda i, j: (j, i)),
        out_shape=jax.ShapeDtypeStruct((_KD, BATCH), jnp.float32),
    )(x)


def _unlines(lines):
    x2 = _tc_transpose(lines.reshape(BATCH, _KD))
    return jnp.transpose(x2).reshape(BATCH, SEQ_LEN, EMB)


def kernel(seqs, exps, table_seq, table_exp):
    lines1 = _sc_lookup_one(_pair_idx(seqs), _pair_table(table_seq))
    lines2 = _sc_lookup_one(_pair_idx(exps), _pair_table(table_exp))
    return (_unlines(lines1), _unlines(lines2))


# speedup vs baseline: 3.1230x; 2.2928x over previous
import functools

import jax
import jax.numpy as jnp
from jax import lax
from jax.experimental import pallas as pl
from jax.experimental.pallas import tpu as pltpu
from jax.experimental.pallas import tpu_sc as plsc

VOCAB = 457
EMB = 64
BATCH = 16384
SEQ_LEN = 200
NPAIR = SEQ_LEN // 2
N = BATCH * SEQ_LEN
LINES = N // 2
PIDX_ROWS = LINES // 128

_info = plsc.get_sparse_core_info()
NC = _info.num_cores
NS = _info.num_subcores
NW = NC * NS
SUB = 128
NSUB = 2
CHUNK_LINES = SUB * NSUB
PER_W_LINES = LINES // NW
N_ITERS = PER_W_LINES // CHUNK_LINES

assert LINES % (NW * CHUNK_LINES) == 0 and N_ITERS % 2 == 0


def _sc_lookup_one(pidx, pair_table):
    mesh = plsc.VectorSubcoreMesh(core_axis_name="c", subcore_axis_name="s")

    @functools.partial(
        pl.kernel,
        mesh=mesh,
        out_type=jax.ShapeDtypeStruct((LINES, 128), jnp.float32),
        scratch_types=[
            pltpu.VMEM((2, 8, SUB), jnp.int32),
            pltpu.VMEM((2, CHUNK_LINES, 128), jnp.float32),
            [pltpu.SemaphoreType.DMA, pltpu.SemaphoreType.DMA],
            [pltpu.SemaphoreType.DMA, pltpu.SemaphoreType.DMA],
        ],
        compiler_params=pltpu.CompilerParams(use_tc_tiling_on_sc=True),
    )
    def k(idx_hbm, tab_hbm, out_hbm, idx_v, rows_v, gsem, osem):
        wid = lax.axis_index("s") * NC + lax.axis_index("c")
        base_irow = wid * (PER_W_LINES // SUB)
        base_line = wid * PER_W_LINES

        def fire_gathers(g, b):
            irow = base_irow + g * NSUB
            pltpu.sync_copy(idx_hbm.at[pl.ds(irow, NSUB)],
                            idx_v.at[b, pl.ds(0, NSUB)])
            for j in range(NSUB):
                pltpu.async_copy(
                    tab_hbm.at[idx_v.at[b, j]],
                    rows_v.at[b, pl.ds(j * SUB, SUB)],
                    gsem[b],
                )

        def wait_gathers(b):
            for j in range(NSUB):
                pltpu.make_async_copy(
                    tab_hbm.at[idx_v.at[b, j]],
                    rows_v.at[b, pl.ds(j * SUB, SUB)],
                    gsem[b],
                ).wait()

        def fire_out(g, b):
            line = base_line + g * CHUNK_LINES
            pltpu.async_copy(rows_v.at[b],
                             out_hbm.at[pl.ds(line, CHUNK_LINES)], osem[b])

        def wait_out(g, b):
            line = base_line + g * CHUNK_LINES
            pltpu.make_async_copy(rows_v.at[b],
                                  out_hbm.at[pl.ds(line, CHUNK_LINES)],
                                  osem[b]).wait()

        fire_gathers(0, 0)

        def step(g2, carry):
            for b in range(2):
                g = 2 * g2 + b
                nb2 = 1 - b

                @pl.when(g + 1 < N_ITERS)
                def _():
                    @pl.when(g >= 1)
                    def _():
                        wait_out(g - 1, nb2)
                    fire_gathers(g + 1, nb2)

                wait_gathers(b)
                fire_out(g, b)
            return carry

        lax.fori_loop(0, N_ITERS // 2, step, 0)
        wait_out(N_ITERS - 1, (N_ITERS - 1) % 2)
        wait_out(N_ITERS - 2, (N_ITERS - 2) % 2)

    return k(pidx, pair_table)


VOCAB_P = 464


def _pair_idx(a):
    z = jnp.transpose(a.astype(jnp.int32)).reshape(NPAIR, 2, BATCH)
    pt = z[:, 0, :] * VOCAB_P + z[:, 1, :]
    return pt.reshape(PIDX_ROWS, 128)


def _pair_table(tab):
    def body(tab_ref, out_ref):
        i = pl.program_id(0)
        left = jnp.broadcast_to(tab_ref[pl.ds(i, 1), :], (VOCAB_P, EMB))
        right = jnp.concatenate(
            [tab_ref[...], jnp.zeros((VOCAB_P - VOCAB, EMB), jnp.float32)], 0)
        out_ref[...] = jnp.concatenate([left, right], axis=1)

    return pl.pallas_call(
        body,
        grid=(VOCAB,),
        in_specs=[pl.BlockSpec((VOCAB, EMB), lambda i: (0, 0))],
        out_specs=pl.BlockSpec((VOCAB_P, 128), lambda i: (i, 0)),
        out_shape=jax.ShapeDtypeStruct((VOCAB * VOCAB_P, 128), jnp.float32),
    )(tab)


_TR_BB = 4096


def _tc_transpose(lt):
    def body(x_ref, y_ref):
        y_ref[...] = x_ref[0].T

    return pl.pallas_call(
        body,
        grid=(NPAIR, BATCH // _TR_BB),
        in_specs=[pl.BlockSpec((1, _TR_BB, 128), lambda g, i: (g, i, 0))],
        out_specs=pl.BlockSpec((128, _TR_BB), lambda g, i: (g, i)),
        out_shape=jax.ShapeDtypeStruct((SEQ_LEN * EMB, BATCH), jnp.float32),
    )(lt.reshape(NPAIR, BATCH, 128))


def _unlines(lines):
    x2 = _tc_transpose(lines)
    return jnp.transpose(x2).reshape(BATCH, SEQ_LEN, EMB)


def kernel(seqs, exps, table_seq, table_exp):
    lines1 = _sc_lookup_one(_pair_idx(seqs), _pair_table(table_seq))
    lines2 = _sc_lookup_one(_pair_idx(exps), _pair_table(table_exp))
    return (_unlines(lines1), _unlines(lines2))


# speedup vs baseline: 3.3238x; 1.0643x over previous
import functools

import jax
import jax.numpy as jnp
from jax import lax
from jax.experimental import pallas as pl
from jax.experimental.pallas import tpu as pltpu
from jax.experimental.pallas import tpu_sc as plsc

VOCAB = 457
EMB = 64
BATCH = 16384
SEQ_LEN = 200
NPAIR = SEQ_LEN // 2
N = BATCH * SEQ_LEN
LINES = N // 2
PIDX_ROWS = LINES // 128

_info = plsc.get_sparse_core_info()
NC = _info.num_cores
NS = _info.num_subcores
NW = NC * NS
SUB = 128
NSUB = 2
CHUNK_LINES = SUB * NSUB
PER_W_LINES = LINES // NW
N_ITERS = PER_W_LINES // CHUNK_LINES

assert LINES % (NW * CHUNK_LINES) == 0 and N_ITERS % 2 == 0


def _sc_lookup_one(pidx, pair_table):
    mesh = plsc.VectorSubcoreMesh(core_axis_name="c", subcore_axis_name="s")

    @functools.partial(
        pl.kernel,
        mesh=mesh,
        out_type=jax.ShapeDtypeStruct((LINES, 128), jnp.float32),
        scratch_types=[
            pltpu.VMEM((2, 8, SUB), jnp.int32),
            pltpu.VMEM((2, CHUNK_LINES, 128), jnp.float32),
            [pltpu.SemaphoreType.DMA, pltpu.SemaphoreType.DMA],
            [pltpu.SemaphoreType.DMA, pltpu.SemaphoreType.DMA],
        ],
        compiler_params=pltpu.CompilerParams(use_tc_tiling_on_sc=True),
    )
    def k(idx_hbm, tab_hbm, out_hbm, idx_v, rows_v, gsem, osem):
        wid = lax.axis_index("s") * NC + lax.axis_index("c")
        base_irow = wid * (PER_W_LINES // SUB)
        base_line = wid * PER_W_LINES

        def fire_gathers(g, b):
            irow = base_irow + g * NSUB
            pltpu.sync_copy(idx_hbm.at[pl.ds(irow, NSUB)],
                            idx_v.at[b, pl.ds(0, NSUB)])
            for j in range(NSUB):
                pltpu.async_copy(
                    tab_hbm.at[idx_v.at[b, j]],
                    rows_v.at[b, pl.ds(j * SUB, SUB)],
                    gsem[b],
                )

        def wait_gathers(b):
            for j in range(NSUB):
                pltpu.make_async_copy(
                    tab_hbm.at[idx_v.at[b, j]],
                    rows_v.at[b, pl.ds(j * SUB, SUB)],
                    gsem[b],
                ).wait()

        def fire_out(g, b):
            line = base_line + g * CHUNK_LINES
            pltpu.async_copy(rows_v.at[b],
                             out_hbm.at[pl.ds(line, CHUNK_LINES)], osem[b])

        def wait_out(g, b):
            line = base_line + g * CHUNK_LINES
            pltpu.make_async_copy(rows_v.at[b],
                                  out_hbm.at[pl.ds(line, CHUNK_LINES)],
                                  osem[b]).wait()

        fire_gathers(0, 0)

        def step(g2, carry):
            for b in range(2):
                g = 2 * g2 + b
                nb2 = 1 - b

                @pl.when(g + 1 < N_ITERS)
                def _():
                    @pl.when(g >= 1)
                    def _():
                        wait_out(g - 1, nb2)
                    fire_gathers(g + 1, nb2)

                wait_gathers(b)
                fire_out(g, b)
            return carry

        lax.fori_loop(0, N_ITERS // 2, step, 0)
        wait_out(N_ITERS - 1, (N_ITERS - 1) % 2)
        wait_out(N_ITERS - 2, (N_ITERS - 2) % 2)

    return k(pidx, pair_table)


VOCAB_P = 464


def _pair_idx(a):
    z = jnp.transpose(a.astype(jnp.int32)).reshape(NPAIR, 2, BATCH)
    pt = z[:, 0, :] * VOCAB_P + z[:, 1, :]
    return pt.reshape(PIDX_ROWS, 128)


_PT_GROUP = 8


def _pair_table(tab):
    def body(tab_ref, out_ref):
        i8 = pl.program_id(0)
        right = jnp.concatenate(
            [tab_ref[...], jnp.zeros((VOCAB_P - VOCAB, EMB), jnp.float32)], 0)
        for ii in range(_PT_GROUP):
            i = jnp.minimum(i8 * _PT_GROUP + ii, VOCAB - 1)
            left = jnp.broadcast_to(tab_ref[pl.ds(i, 1), :], (VOCAB_P, EMB))
            out_ref[pl.ds(ii * VOCAB_P, VOCAB_P), :] = jnp.concatenate(
                [left, right], axis=1)

    return pl.pallas_call(
        body,
        grid=(VOCAB_P // _PT_GROUP,),
        in_specs=[pl.BlockSpec((VOCAB, EMB), lambda i: (0, 0))],
        out_specs=pl.BlockSpec((_PT_GROUP * VOCAB_P, 128), lambda i: (i, 0)),
        out_shape=jax.ShapeDtypeStruct((VOCAB_P * VOCAB_P, 128), jnp.float32),
    )(tab)


_TR_BB = 4096


def _tc_transpose(lt):
    def body(x_ref, y_ref):
        y_ref[...] = x_ref[0].T

    return pl.pallas_call(
        body,
        grid=(NPAIR, BATCH // _TR_BB),
        in_specs=[pl.BlockSpec((1, _TR_BB, 128), lambda g, i: (g, i, 0))],
        out_specs=pl.BlockSpec((128, _TR_BB), lambda g, i: (g, i)),
        out_shape=jax.ShapeDtypeStruct((SEQ_LEN * EMB, BATCH), jnp.float32),
    )(lt.reshape(NPAIR, BATCH, 128))


def _unlines(lines):
    x2 = _tc_transpose(lines)
    return jnp.transpose(x2).reshape(BATCH, SEQ_LEN, EMB)


def kernel(seqs, exps, table_seq, table_exp):
    lines1 = _sc_lookup_one(_pair_idx(seqs), _pair_table(table_seq))
    lines2 = _sc_lookup_one(_pair_idx(exps), _pair_table(table_exp))
    return (_unlines(lines1), _unlines(lines2))


# speedup vs baseline: 4.2314x; 1.2731x over previous
import functools

import jax
import jax.numpy as jnp
from jax import lax
from jax.experimental import pallas as pl
from jax.experimental.pallas import tpu as pltpu
from jax.experimental.pallas import tpu_sc as plsc

VOCAB = 457
EMB = 64
BATCH = 16384
SEQ_LEN = 200
NPAIR = SEQ_LEN // 2
N = BATCH * SEQ_LEN
LINES = N // 2
IDX4_ROWS = N // 128

_info = plsc.get_sparse_core_info()
NC = _info.num_cores
NS = _info.num_subcores
NW = NC * NS
SUB = 128
CHUNK_LINES = 256
PER_W_LINES = LINES // NW
N_ITERS = PER_W_LINES // CHUNK_LINES

assert LINES % (NW * CHUNK_LINES) == 0 and N_ITERS % 2 == 0


def _sc_lookup_one(idx4, table):
    mesh = plsc.VectorSubcoreMesh(core_axis_name="c", subcore_axis_name="s")

    @functools.partial(
        pl.kernel,
        mesh=mesh,
        out_type=jax.ShapeDtypeStruct((LINES, 128), jnp.float32),
        scratch_types=[
            pltpu.VMEM_SHARED((VOCAB, EMB), jnp.float32),
            pltpu.VMEM((2, 4, SUB), jnp.int32),
            pltpu.VMEM((2, 4, SUB, EMB), jnp.float32),
            [pltpu.SemaphoreType.DMA, pltpu.SemaphoreType.DMA],
            [pltpu.SemaphoreType.DMA, pltpu.SemaphoreType.DMA],
        ],
        compiler_params=pltpu.CompilerParams(use_tc_tiling_on_sc=False),
    )
    def k(idx_hbm, tab_hbm, out_hbm, tab_v, idx_v, rows_v, gsem, osem):
        wid = lax.axis_index("s") * NC + lax.axis_index("c")
        base_irow = wid * 4 * N_ITERS
        base_line = wid * PER_W_LINES

        @pl.when(lax.axis_index("s") == 0)
        def _():
            pltpu.sync_copy(tab_hbm, tab_v)

        plsc.subcore_barrier()

        def fire_gathers(g, b):
            irow = base_irow + g * 4
            pltpu.sync_copy(idx_hbm.at[pl.ds(irow, 4)], idx_v.at[b])
            for j in range(4):
                pltpu.async_copy(
                    tab_v.at[idx_v.at[b, j]],
                    rows_v.at[b, j],
                    gsem[b],
                )

        def wait_gathers(b):
            for j in range(4):
                pltpu.make_async_copy(
                    tab_v.at[idx_v.at[b, j]],
                    rows_v.at[b, j],
                    gsem[b],
                ).wait()

        def _out_slices(g):
            line = base_line + g * CHUNK_LINES
            return [
                out_hbm.at[pl.ds(line, SUB), pl.ds(0, EMB)],
                out_hbm.at[pl.ds(line + SUB, SUB), pl.ds(0, EMB)],
                out_hbm.at[pl.ds(line, SUB), pl.ds(EMB, EMB)],
                out_hbm.at[pl.ds(line + SUB, SUB), pl.ds(EMB, EMB)],
            ]

        def fire_out(g, b):
            for j, dst in enumerate(_out_slices(g)):
                pltpu.async_copy(rows_v.at[b, j], dst, osem[b])

        def wait_out(g, b):
            for j, dst in enumerate(_out_slices(g)):
                pltpu.make_async_copy(rows_v.at[b, j], dst, osem[b]).wait()

        fire_gathers(0, 0)

        def step(g2, carry):
            for b in range(2):
                g = 2 * g2 + b
                nb2 = 1 - b

                @pl.when(g + 1 < N_ITERS)
                def _():
                    @pl.when(g >= 1)
                    def _():
                        wait_out(g - 1, nb2)
                    fire_gathers(g + 1, nb2)

                wait_gathers(b)
                fire_out(g, b)
            return carry

        lax.fori_loop(0, N_ITERS // 2, step, 0)
        wait_out(N_ITERS - 1, (N_ITERS - 1) % 2)
        wait_out(N_ITERS - 2, (N_ITERS - 2) % 2)

    return k(idx4, table)


def _idx4(a):
    z = jnp.transpose(a.astype(jnp.int32)).reshape(NPAIR, 2, BATCH)
    ev = z[:, 0, :].reshape(IDX4_ROWS // 4, 2, 128)
    od = z[:, 1, :].reshape(IDX4_ROWS // 4, 2, 128)
    return jnp.stack([ev, od], axis=1).reshape(IDX4_ROWS, 128)


_TR_BB = 4096


def _tc_transpose(lt):
    def body(x_ref, y_ref):
        y_ref[...] = x_ref[0].T

    return pl.pallas_call(
        body,
        grid=(NPAIR, BATCH // _TR_BB),
        in_specs=[pl.BlockSpec((1, _TR_BB, 128), lambda g, i: (g, i, 0))],
        out_specs=pl.BlockSpec((128, _TR_BB), lambda g, i: (g, i)),
        out_shape=jax.ShapeDtypeStruct((SEQ_LEN * EMB, BATCH), jnp.float32),
    )(lt.reshape(NPAIR, BATCH, 128))


def _unlines(lines):
    x2 = _tc_transpose(lines)
    return jnp.transpose(x2).reshape(BATCH, SEQ_LEN, EMB)


def kernel(seqs, exps, table_seq, table_exp):
    lines1 = _sc_lookup_one(_idx4(seqs), table_seq)
    lines2 = _sc_lookup_one(_idx4(exps), table_exp)
    return (_unlines(lines1), _unlines(lines2))
